# Initial kernel scaffold; baseline (speedup 1.0000x reference)
import jax
import jax.numpy as jnp
from jax.experimental import pallas as pl


def kernel(edge_index, edge_type, node_embeds, basis, comp, root, bias, special):
    raise NotImplementedError("write your pallas kernel here")


# speedup vs baseline: 2.7501x; 2.7501x over previous
import functools

import jax
import jax.numpy as jnp
from jax import lax
from jax.experimental import pallas as pl
from jax.experimental.pallas import tpu as pltpu
from jax.experimental.pallas import tpu_sc as plsc

NC = 2
NS = 16
L = 16
CH = 128


def _make_allh(R, NB, N, H, NBLK):
    def w_body(comp_ref, basis_ref, out_ref):
        w = jnp.dot(
            comp_ref[...],
            basis_ref[...].reshape(NB, H * H),
            preferred_element_type=jnp.float32,
        )
        out_ref[...] = w.reshape(R, H, H)

    def allh_body(w_ref, x_ref, out_ref):
        out_ref[0] = jnp.dot(x_ref[...], w_ref[0], preferred_element_type=jnp.float32)

    def run(comp, basis, x):
        weight = pl.pallas_call(
            w_body,
            out_shape=jax.ShapeDtypeStruct((R, H, H), jnp.float32),
        )(comp, basis)
        all_h = pl.pallas_call(
            allh_body,
            grid=(N // NBLK, R),
            in_specs=[
                pl.BlockSpec((1, H, H), lambda n, r: (r, 0, 0)),
                pl.BlockSpec((NBLK, H), lambda n, r: (n, 0)),
            ],
            out_specs=pl.BlockSpec((1, NBLK, H), lambda n, r: (r, n, 0)),
            out_shape=jax.ShapeDtypeStruct((R, N, H), jnp.float32),
        )(weight, x)
        return all_h.reshape(R * N, H)

    return run


def _make_sc_edge(E, N, R, H):
    nchunks = E // CH
    assert E % CH == 0
    cnt_size = N * R
    b1, r1 = divmod(nchunks, NS)
    NW = NC * NS
    b2, r2 = divmod(nchunks, NW)
    AU = 80
    assert N % AU == 0
    na, ra = divmod(N // AU, NS)
    CU = 3840
    assert cnt_size % CU == 0
    nc_, rc_ = divmod(cnt_size // CU, NS)

    mesh = plsc.VectorSubcoreMesh(core_axis_name="c", subcore_axis_name="s")

    @functools.partial(
        pl.kernel,
        out_type=jax.ShapeDtypeStruct((NC, N, H), jnp.float32),
        mesh=mesh,
        scratch_types=[
            pltpu.VMEM((CH,), jnp.int32),
            pltpu.VMEM((CH,), jnp.int32),
            pltpu.VMEM((CH,), jnp.int32),
            pltpu.VMEM((CH,), jnp.int32),
            pltpu.VMEM((CH,), jnp.int32),
            pltpu.VMEM((CH,), jnp.float32),
            pltpu.VMEM((CH,), jnp.float32),
            pltpu.VMEM((CH, H), jnp.float32),
            pltpu.VMEM_SHARED((cnt_size,), jnp.float32),
            pltpu.VMEM_SHARED((N, H), jnp.float32),
            pltpu.SemaphoreType.DMA,
        ],
    )
    def sc_edge(ei, et, allh, zrow, zcnt, out,
                typ_v, dst_v, src_v, pair_v, midx_v, den_v, ones_v, rows_v,
                cnt_sh, agg_sh, sem):
        c = lax.axis_index("c")
        s = lax.axis_index("s")
        w = s * NC + c

        def zc_body(i, carry):
            pltpu.sync_copy(zcnt, cnt_sh.at[pl.ds((s + i * NS) * CU, CU)])
            return carry

        lax.fori_loop(0, nc_ + jnp.where(s < rc_, 1, 0), zc_body, 0)

        def za_body(i, carry):
            pltpu.sync_copy(zrow, agg_sh.at[pl.ds((s + i * NS) * AU, AU)])
            return carry

        lax.fori_loop(0, na + jnp.where(s < ra, 1, 0), za_body, 0)
        for j in range(CH // L):
            ones_v[pl.ds(j * L, L)] = jnp.full((L,), 1.0, jnp.float32)
        plsc.subcore_barrier()

        n1 = b1 + jnp.where(s < r1, 1, 0)

        def p1_body(i, carry):
            off = (s + i * NS) * CH
            pltpu.sync_copy(et.at[pl.ds(off, CH)], typ_v)
            pltpu.sync_copy(ei.at[1, pl.ds(off, CH)], dst_v)
            for j in range(CH // L):
                sl = pl.ds(j * L, L)
                pair_v[sl] = dst_v[sl] * R + typ_v[sl]
            pltpu.sync_copy(ones_v, cnt_sh.at[pair_v], add=True)
            return carry

        lax.fori_loop(0, n1, p1_body, 0)
        plsc.subcore_barrier()

        n2 = b2 + jnp.where(w < r2, 1, 0)

        def p2_body(i, carry):
            off = (w + i * NW) * CH
            pltpu.sync_copy(et.at[pl.ds(off, CH)], typ_v)
            pltpu.sync_copy(ei.at[0, pl.ds(off, CH)], src_v)
            pltpu.sync_copy(ei.at[1, pl.ds(off, CH)], dst_v)
            for j in range(CH // L):
                sl = pl.ds(j * L, L)
                t = typ_v[sl]
                pair_v[sl] = dst_v[sl] * R + t
                midx_v[sl] = t * N + src_v[sl]
            pltpu.sync_copy(cnt_sh.at[pair_v], den_v)
            cp = pltpu.async_copy(allh.at[midx_v], rows_v, sem)
            for j in range(CH // L):
                sl = pl.ds(j * L, L)
                den_v[sl] = 1.0 / jnp.maximum(den_v[sl], 1.0)
            cp.wait()

            def scale_body(g, cc):
                den16 = den_v[pl.ds(g * L, L)]
                for e in range(L):
                    sc = den16[e]
                    row = g * L + e
                    for j in range(H // L):
                        sl = pl.ds(j * L, L)
                        rows_v[row, sl] = rows_v[row, sl] * sc
                return cc

            lax.fori_loop(0, CH // L, scale_body, 0)
            pltpu.sync_copy(rows_v, agg_sh.at[dst_v], add=True)
            return carry

        lax.fori_loop(0, n2, p2_body, 0)
        plsc.subcore_barrier()

        def p3_body(i, carry):
            r0 = (s + i * NS) * AU
            pltpu.sync_copy(agg_sh.at[pl.ds(r0, AU)], rows_v.at[pl.ds(0, AU)])
            pltpu.sync_copy(rows_v.at[pl.ds(0, AU)], out.at[c, pl.ds(r0, AU)])
            return carry

        lax.fori_loop(0, na + jnp.where(s < ra, 1, 0), p3_body, 0)

    return sc_edge


def _make_combine(N, H, NBLK):
    def body(agg_ref, x_ref, root_ref, bias_ref, o_ref):
        o_ref[...] = (
            agg_ref[0]
            + agg_ref[1]
            + jnp.dot(x_ref[...], root_ref[...], preferred_element_type=jnp.float32)
            + bias_ref[...]
        )

    def run(agg, x, root, bias):
        return pl.pallas_call(
            body,
            grid=(N // NBLK,),
            in_specs=[
                pl.BlockSpec((NC, NBLK, H), lambda n: (0, n, 0)),
                pl.BlockSpec((NBLK, H), lambda n: (n, 0)),
                pl.BlockSpec((H, H), lambda n: (0, 0)),
                pl.BlockSpec((1, H), lambda n: (0, 0)),
            ],
            out_specs=pl.BlockSpec((NBLK, H), lambda n: (n, 0)),
            out_shape=jax.ShapeDtypeStruct((N, H), jnp.float32),
        )(agg, x, root, bias.reshape(1, H))

    return run


def kernel(edge_index, edge_type, node_embeds, basis, comp, root, bias, special):
    N, H = node_embeds.shape
    R, NB = comp.shape
    E = edge_type.shape[0]
    NBLK = 2000

    all_h = _make_allh(R, NB, N, H, NBLK)(comp, basis, node_embeds)

    zrow = jnp.zeros((80, H), jnp.float32)
    zcnt = jnp.zeros((3840,), jnp.float32)
    agg = _make_sc_edge(E, N, R, H)(edge_index, edge_type, all_h, zrow, zcnt)

    node_out = _make_combine(N, H, NBLK)(agg, node_embeds, root, bias)
    return jnp.concatenate([node_out, special], axis=0)


# speedup vs baseline: 3.4147x; 1.2417x over previous
import functools

import jax
import jax.numpy as jnp
from jax import lax
from jax.experimental import pallas as pl
from jax.experimental.pallas import tpu as pltpu
from jax.experimental.pallas import tpu_sc as plsc

NC = 2
NS = 16
L = 16
CH = 128


def _make_allh(R, NB, N, H, NBLK):
    def w_body(comp_ref, basis_ref, out_ref):
        w = jnp.dot(
            comp_ref[...],
            basis_ref[...].reshape(NB, H * H),
            preferred_element_type=jnp.float32,
        )
        out_ref[...] = w.reshape(R, H, H)

    def allh_body(w_ref, x_ref, out_ref):
        out_ref[0] = jnp.dot(x_ref[...], w_ref[0], preferred_element_type=jnp.float32)

    def run(comp, basis, x):
        weight = pl.pallas_call(
            w_body,
            out_shape=jax.ShapeDtypeStruct((R, H, H), jnp.float32),
        )(comp, basis)
        all_h = pl.pallas_call(
            allh_body,
            grid=(N // NBLK, R),
            in_specs=[
                pl.BlockSpec((1, H, H), lambda n, r: (r, 0, 0)),
                pl.BlockSpec((NBLK, H), lambda n, r: (n, 0)),
            ],
            out_specs=pl.BlockSpec((1, NBLK, H), lambda n, r: (r, n, 0)),
            out_shape=jax.ShapeDtypeStruct((R, N, H), jnp.float32),
        )(weight, x)
        return all_h.reshape(R * N, H)

    return run


def _make_sc_count(E, N, R):
    nchunks = E // CH
    assert E % CH == 0
    cnt_size = N * R
    half = nchunks // NC
    assert nchunks % NC == 0
    b1, r1 = divmod(half, NS)
    CU = 3840
    assert cnt_size % CU == 0
    nc_, rc_ = divmod(cnt_size // CU, NS)

    mesh = plsc.VectorSubcoreMesh(core_axis_name="c", subcore_axis_name="s")

    @functools.partial(
        pl.kernel,
        out_type=jax.ShapeDtypeStruct((NC, cnt_size), jnp.float32),
        mesh=mesh,
        scratch_types=[
            pltpu.VMEM((CH,), jnp.int32),
            pltpu.VMEM((CH,), jnp.int32),
            pltpu.VMEM((CH,), jnp.int32),
            pltpu.VMEM((CH,), jnp.float32),
            pltpu.VMEM((CU,), jnp.float32),
            pltpu.VMEM_SHARED((cnt_size,), jnp.float32),
        ],
    )
    def sc_count(ei, et, zcnt, out, typ_v, dst_v, pair_v, ones_v, bounce_v, cnt_sh):
        c = lax.axis_index("c")
        s = lax.axis_index("s")

        def zc_body(i, carry):
            pltpu.sync_copy(zcnt, cnt_sh.at[pl.ds((s + i * NS) * CU, CU)])
            return carry

        lax.fori_loop(0, nc_ + jnp.where(s < rc_, 1, 0), zc_body, 0)
        for j in range(CH // L):
            ones_v[pl.ds(j * L, L)] = jnp.full((L,), 1.0, jnp.float32)
        plsc.subcore_barrier()

        n1 = b1 + jnp.where(s < r1, 1, 0)

        def p1_body(i, carry):
            off = (c * half + s + i * NS) * CH
            pltpu.sync_copy(et.at[pl.ds(off, CH)], typ_v)
            pltpu.sync_copy(ei.at[1, pl.ds(off, CH)], dst_v)
            for j in range(CH // L):
                sl = pl.ds(j * L, L)
                pair_v[sl] = dst_v[sl] * R + typ_v[sl]
            pltpu.sync_copy(ones_v, cnt_sh.at[pair_v], add=True)
            return carry

        lax.fori_loop(0, n1, p1_body, 0)
        plsc.subcore_barrier()

        def cp_body(i, carry):
            o = (s + i * NS) * CU
            pltpu.sync_copy(cnt_sh.at[pl.ds(o, CU)], bounce_v)
            pltpu.sync_copy(bounce_v, out.at[c, pl.ds(o, CU)])
            return carry

        lax.fori_loop(0, nc_ + jnp.where(s < rc_, 1, 0), cp_body, 0)

    return sc_count


def _make_sc_edge(E, N, R, H):
    nchunks = E // CH
    assert E % CH == 0
    NW = NC * NS
    b2, r2 = divmod(nchunks, NW)
    AU = 80
    assert N % AU == 0
    na, ra = divmod(N // AU, NS)

    mesh = plsc.VectorSubcoreMesh(core_axis_name="c", subcore_axis_name="s")

    @functools.partial(
        pl.kernel,
        out_type=jax.ShapeDtypeStruct((NC, N, H), jnp.float32),
        mesh=mesh,
        scratch_types=[
            pltpu.VMEM((CH,), jnp.int32),
            pltpu.VMEM((CH,), jnp.int32),
            pltpu.VMEM((CH,), jnp.int32),
            pltpu.VMEM((CH,), jnp.int32),
            pltpu.VMEM((CH,), jnp.int32),
            pltpu.VMEM((CH,), jnp.float32),
            pltpu.VMEM((CH,), jnp.float32),
            pltpu.VMEM((CH, H), jnp.float32),
            pltpu.VMEM_SHARED((N, H), jnp.float32),
            pltpu.SemaphoreType.DMA,
            pltpu.SemaphoreType.DMA,
            pltpu.SemaphoreType.DMA,
        ],
    )
    def sc_edge(ei, et, allh, cnt0, cnt1, zrow, out,
                typ_v, dst_v, src_v, pair_v, midx_v, den_v, den2_v, rows_v,
                agg_sh, sem, sem2, sem3):
        c = lax.axis_index("c")
        s = lax.axis_index("s")
        w = s * NC + c

        def za_body(i, carry):
            pltpu.sync_copy(zrow, agg_sh.at[pl.ds((s + i * NS) * AU, AU)])
            return carry

        lax.fori_loop(0, na + jnp.where(s < ra, 1, 0), za_body, 0)
        plsc.subcore_barrier()

        n2 = b2 + jnp.where(w < r2, 1, 0)

        def p2_body(i, carry):
            off = (w + i * NW) * CH
            pltpu.sync_copy(et.at[pl.ds(off, CH)], typ_v)
            pltpu.sync_copy(ei.at[0, pl.ds(off, CH)], src_v)
            pltpu.sync_copy(ei.at[1, pl.ds(off, CH)], dst_v)
            for j in range(CH // L):
                sl = pl.ds(j * L, L)
                t = typ_v[sl]
                pair_v[sl] = dst_v[sl] * R + t
                midx_v[sl] = t * N + src_v[sl]
            cp = pltpu.async_copy(allh.at[midx_v], rows_v, sem)
            cd0 = pltpu.async_copy(cnt0.at[pair_v], den_v, sem2)
            cd1 = pltpu.async_copy(cnt1.at[pair_v], den2_v, sem3)
            cd0.wait()
            cd1.wait()
            for j in range(CH // L):
                sl = pl.ds(j * L, L)
                den_v[sl] = 1.0 / jnp.maximum(den_v[sl] + den2_v[sl], 1.0)
            cp.wait()

            def scale_body(g, cc):
                den16 = den_v[pl.ds(g * L, L)]
                for e in range(L):
                    sc = den16[e]
                    row = g * L + e
                    for j in range(H // L):
                        sl = pl.ds(j * L, L)
                        rows_v[row, sl] = rows_v[row, sl] * sc
                return cc

            lax.fori_loop(0, CH // L, scale_body, 0)
            pltpu.sync_copy(rows_v, agg_sh.at[dst_v], add=True)
            return carry

        lax.fori_loop(0, n2, p2_body, 0)
        plsc.subcore_barrier()

        def p3_body(i, carry):
            r0 = (s + i * NS) * AU
            pltpu.sync_copy(agg_sh.at[pl.ds(r0, AU)], rows_v.at[pl.ds(0, AU)])
            pltpu.sync_copy(rows_v.at[pl.ds(0, AU)], out.at[c, pl.ds(r0, AU)])
            return carry

        lax.fori_loop(0, na + jnp.where(s < ra, 1, 0), p3_body, 0)

    return sc_edge


def _make_combine(N, H, NBLK):
    def body(agg_ref, x_ref, root_ref, bias_ref, o_ref):
        o_ref[...] = (
            agg_ref[0]
            + agg_ref[1]
            + jnp.dot(x_ref[...], root_ref[...], preferred_element_type=jnp.float32)
            + bias_ref[...]
        )

    def run(agg, x, root, bias):
        return pl.pallas_call(
            body,
            grid=(N // NBLK,),
            in_specs=[
                pl.BlockSpec((NC, NBLK, H), lambda n: (0, n, 0)),
                pl.BlockSpec((NBLK, H), lambda n: (n, 0)),
                pl.BlockSpec((H, H), lambda n: (0, 0)),
                pl.BlockSpec((1, H), lambda n: (0, 0)),
            ],
            out_specs=pl.BlockSpec((NBLK, H), lambda n: (n, 0)),
            out_shape=jax.ShapeDtypeStruct((N, H), jnp.float32),
        )(agg, x, root, bias.reshape(1, H))

    return run


def kernel(edge_index, edge_type, node_embeds, basis, comp, root, bias, special):
    N, H = node_embeds.shape
    R, NB = comp.shape
    E = edge_type.shape[0]
    NBLK = 2000

    zcnt = jnp.zeros((3840,), jnp.float32)
    cnt2 = _make_sc_count(E, N, R)(edge_index, edge_type, zcnt)

    all_h = _make_allh(R, NB, N, H, NBLK)(comp, basis, node_embeds)

    zrow = jnp.zeros((80, H), jnp.float32)
    agg = _make_sc_edge(E, N, R, H)(
        edge_index, edge_type, all_h, cnt2[0], cnt2[1], zrow
    )

    node_out = _make_combine(N, H, NBLK)(agg, node_embeds, root, bias)
    return jnp.concatenate([node_out, special], axis=0)


# speedup vs baseline: 4.1961x; 1.2288x over previous
import functools

import jax
import jax.numpy as jnp
from jax import lax
from jax.experimental import pallas as pl
from jax.experimental.pallas import tpu as pltpu
from jax.experimental.pallas import tpu_sc as plsc

NC = 2
NS = 16
L = 16
CH = 128


def _make_allh(R, NB, N, H, NBLK):
    def w_body(comp_ref, basis_ref, out_ref):
        w = jnp.dot(
            comp_ref[...],
            basis_ref[...].reshape(NB, H * H),
            preferred_element_type=jnp.float32,
        )
        out_ref[...] = w.reshape(R, H, H)

    def allh_body(w_ref, x_ref, out_ref):
        out_ref[0] = jnp.dot(x_ref[...], w_ref[0], preferred_element_type=jnp.float32)

    def run(comp, basis, x):
        weight = pl.pallas_call(
            w_body,
            out_shape=jax.ShapeDtypeStruct((R, H, H), jnp.float32),
        )(comp, basis)
        all_h = pl.pallas_call(
            allh_body,
            grid=(N // NBLK, R),
            in_specs=[
                pl.BlockSpec((1, H, H), lambda n, r: (r, 0, 0)),
                pl.BlockSpec((NBLK, H), lambda n, r: (n, 0)),
            ],
            out_specs=pl.BlockSpec((1, NBLK, H), lambda n, r: (r, n, 0)),
            out_shape=jax.ShapeDtypeStruct((R, N, H), jnp.float32),
        )(weight, x)
        return all_h.reshape(R * N, H)

    return run


def _make_sc_count(E, N, R):
    nchunks = E // CH
    assert E % CH == 0
    cnt_size = N * R
    half = nchunks // NC
    assert nchunks % NC == 0
    b1, r1 = divmod(half, NS)
    CU = 3840
    assert cnt_size % CU == 0
    nc_, rc_ = divmod(cnt_size // CU, NS)

    mesh = plsc.VectorSubcoreMesh(core_axis_name="c", subcore_axis_name="s")

    @functools.partial(
        pl.kernel,
        out_type=jax.ShapeDtypeStruct((NC, cnt_size), jnp.float32),
        mesh=mesh,
        scratch_types=[
            pltpu.VMEM((CH,), jnp.int32),
            pltpu.VMEM((CH,), jnp.int32),
            pltpu.VMEM((CH,), jnp.int32),
            pltpu.VMEM((CH,), jnp.float32),
            pltpu.VMEM((CU,), jnp.float32),
            pltpu.VMEM_SHARED((cnt_size,), jnp.float32),
        ],
    )
    def sc_count(ei, et, zcnt, out, typ_v, dst_v, pair_v, ones_v, bounce_v, cnt_sh):
        c = lax.axis_index("c")
        s = lax.axis_index("s")

        def zc_body(i, carry):
            pltpu.sync_copy(zcnt, cnt_sh.at[pl.ds((s + i * NS) * CU, CU)])
            return carry

        lax.fori_loop(0, nc_ + jnp.where(s < rc_, 1, 0), zc_body, 0)
        for j in range(CH // L):
            ones_v[pl.ds(j * L, L)] = jnp.full((L,), 1.0, jnp.float32)
        plsc.subcore_barrier()

        n1 = b1 + jnp.where(s < r1, 1, 0)

        def p1_body(i, carry):
            off = (c * half + s + i * NS) * CH
            pltpu.sync_copy(et.at[pl.ds(off, CH)], typ_v)
            pltpu.sync_copy(ei.at[1, pl.ds(off, CH)], dst_v)
            for j in range(CH // L):
                sl = pl.ds(j * L, L)
                pair_v[sl] = dst_v[sl] * R + typ_v[sl]
            pltpu.sync_copy(ones_v, cnt_sh.at[pair_v], add=True)
            return carry

        lax.fori_loop(0, n1, p1_body, 0)
        plsc.subcore_barrier()

        def cp_body(i, carry):
            o = (s + i * NS) * CU
            pltpu.sync_copy(cnt_sh.at[pl.ds(o, CU)], bounce_v)
            pltpu.sync_copy(bounce_v, out.at[c, pl.ds(o, CU)])
            return carry

        lax.fori_loop(0, nc_ + jnp.where(s < rc_, 1, 0), cp_body, 0)

    return sc_count


def _make_sc_edge(E, N, R, H):
    nchunks = E // CH
    assert E % CH == 0
    NW = NC * NS
    DEPTH = 2
    nmax = -(-nchunks // NW)
    nmax = -(-nmax // DEPTH) * DEPTH
    AU = 80
    assert N % AU == 0
    na, ra = divmod(N // AU, NS)

    mesh = plsc.VectorSubcoreMesh(core_axis_name="c", subcore_axis_name="s")

    def _dma(n):
        return [pltpu.SemaphoreType.DMA for _ in range(n)]

    @functools.partial(
        pl.kernel,
        out_type=jax.ShapeDtypeStruct((NC, N, H), jnp.float32),
        mesh=mesh,
        scratch_types=(
            [pltpu.VMEM((CH,), jnp.int32) for _ in range(2 * DEPTH)]
            + [pltpu.VMEM((2, CH), jnp.int32) for _ in range(DEPTH)]
            + [pltpu.VMEM((CH,), jnp.int32) for _ in range(2 * DEPTH)]
            + [pltpu.VMEM((CH,), jnp.float32) for _ in range(2 * DEPTH)]
            + [pltpu.VMEM((CH, H), jnp.float32) for _ in range(DEPTH)]
            + [pltpu.VMEM_SHARED((N, H), jnp.float32)]
            + _dma(6 * DEPTH)
        ),
    )
    def sc_edge(ei, et, allh, cnt0, cnt1, zrow, out, *scr):
        typ_d = scr[0:DEPTH]
        dst_d = scr[DEPTH:2 * DEPTH]
        ei_d = scr[2 * DEPTH:3 * DEPTH]
        pair_d = scr[3 * DEPTH:4 * DEPTH]
        midx_d = scr[4 * DEPTH:5 * DEPTH]
        den_d = scr[5 * DEPTH:6 * DEPTH]
        den2_d = scr[6 * DEPTH:7 * DEPTH]
        rows_d = scr[7 * DEPTH:8 * DEPTH]
        agg_sh = scr[8 * DEPTH]
        sems = scr[8 * DEPTH + 1:]

        c = lax.axis_index("c")
        s = lax.axis_index("s")
        w = s * NC + c

        def za_body(i, carry):
            pltpu.sync_copy(zrow, agg_sh.at[pl.ds((s + i * NS) * AU, AU)])
            return carry

        lax.fori_loop(0, na + jnp.where(s < ra, 1, 0), za_body, 0)
        plsc.subcore_barrier()

        def p2_body(i, carry):
            ks = [w + (DEPTH * i + d) * NW for d in range(DEPTH)]
            offs = [jnp.minimum(k, nchunks - 1) * CH for k in ks]
            lds = []
            for d in range(DEPTH):
                lds.append((
                    pltpu.async_copy(et.at[pl.ds(offs[d], CH)], typ_d[d],
                                     sems[6 * d]),
                    pltpu.async_copy(ei.at[:, pl.ds(offs[d], CH)], ei_d[d],
                                     sems[6 * d + 1]),
                ))
            gs = []
            for d in range(DEPTH):
                lds[d][0].wait()
                lds[d][1].wait()
                for j in range(CH // L):
                    sl = pl.ds(j * L, L)
                    t = typ_d[d][sl]
                    dv = ei_d[d][1, sl]
                    dst_d[d][sl] = dv
                    pair_d[d][sl] = dv * R + t
                    midx_d[d][sl] = t * N + ei_d[d][0, sl]
                gs.append((
                    pltpu.async_copy(allh.at[midx_d[d]], rows_d[d],
                                     sems[6 * d + 2]),
                    pltpu.async_copy(cnt0.at[pair_d[d]], den_d[d],
                                     sems[6 * d + 3]),
                    pltpu.async_copy(cnt1.at[pair_d[d]], den2_d[d],
                                     sems[6 * d + 4]),
                ))
            for d in range(DEPTH):
                act = jnp.where(ks[d] < nchunks, 1.0, 0.0)
                gs[d][1].wait()
                gs[d][2].wait()
                for j in range(CH // L):
                    sl = pl.ds(j * L, L)
                    den_d[d][sl] = act / jnp.maximum(
                        den_d[d][sl] + den2_d[d][sl], 1.0)
                gs[d][0].wait()

                def scale_body(g, cc, d=d):
                    den16 = den_d[d][pl.ds(g * L, L)]
                    for e in range(L):
                        sc = den16[e]
                        row = g * L + e
                        for j in range(H // L):
                            sl = pl.ds(j * L, L)
                            rows_d[d][row, sl] = rows_d[d][row, sl] * sc
                    return cc

                lax.fori_loop(0, CH // L, scale_body, 0)
                pltpu.sync_copy(rows_d[d], agg_sh.at[dst_d[d]], add=True)
            return carry

        lax.fori_loop(0, nmax // DEPTH, p2_body, 0)
        plsc.subcore_barrier()

        def p3_body(i, carry):
            r0 = (s + i * NS) * AU
            pltpu.sync_copy(agg_sh.at[pl.ds(r0, AU)], rows_d[0].at[pl.ds(0, AU)])
            pltpu.sync_copy(rows_d[0].at[pl.ds(0, AU)], out.at[c, pl.ds(r0, AU)])
            return carry

        lax.fori_loop(0, na + jnp.where(s < ra, 1, 0), p3_body, 0)

    return sc_edge


def _make_combine(N, H, NBLK):
    def body(agg_ref, x_ref, root_ref, bias_ref, o_ref):
        o_ref[...] = (
            agg_ref[0]
            + agg_ref[1]
            + jnp.dot(x_ref[...], root_ref[...], preferred_element_type=jnp.float32)
            + bias_ref[...]
        )

    def run(agg, x, root, bias):
        return pl.pallas_call(
            body,
            grid=(N // NBLK,),
            in_specs=[
                pl.BlockSpec((NC, NBLK, H), lambda n: (0, n, 0)),
                pl.BlockSpec((NBLK, H), lambda n: (n, 0)),
                pl.BlockSpec((H, H), lambda n: (0, 0)),
                pl.BlockSpec((1, H), lambda n: (0, 0)),
            ],
            out_specs=pl.BlockSpec((NBLK, H), lambda n: (n, 0)),
            out_shape=jax.ShapeDtypeStruct((N, H), jnp.float32),
        )(agg, x, root, bias.reshape(1, H))

    return run


def kernel(edge_index, edge_type, node_embeds, basis, comp, root, bias, special):
    N, H = node_embeds.shape
    R, NB = comp.shape
    E = edge_type.shape[0]
    NBLK = 2000

    zcnt = jnp.zeros((3840,), jnp.float32)
    cnt2 = _make_sc_count(E, N, R)(edge_index, edge_type, zcnt)

    all_h = _make_allh(R, NB, N, H, NBLK)(comp, basis, node_embeds)

    zrow = jnp.zeros((80, H), jnp.float32)
    agg = _make_sc_edge(E, N, R, H)(
        edge_index, edge_type, all_h, cnt2[0], cnt2[1], zrow
    )

    node_out = _make_combine(N, H, NBLK)(agg, node_embeds, root, bias)
    return jnp.concatenate([node_out, special], axis=0)
